# initial kernel scaffold (unmeasured)
import jax
import jax.numpy as jnp
from jax import lax
from jax.experimental import pallas as pl
from jax.experimental.pallas import tpu as pltpu

N_DEV = 4
N_TILE = 512


def kernel(x, w_mat):
    m_total, k_shard = x.shape
    k_total, n_total = w_mat.shape
    m_per = m_total // N_DEV
    nt = n_total // N_TILE

    me = lax.axis_index("i")
    order = jnp.stack(
        [me, (me + 1) % N_DEV, (me + 3) % N_DEV, (me + 2) % N_DEV]
    ).astype(jnp.int32)

    def body(pr, x_hbm, x_loc, w_blk, out_ref, recv_buf, send_sems, recv_sems):
        j = pl.program_id(0)
        n = pl.program_id(1)
        my = pr[0]
        s = pr[j]

        @pl.when(jnp.logical_and(j == 0, n == 0))
        def _():
            for d in range(1, N_DEV):
                p = (my + d) % N_DEV
                pltpu.make_async_remote_copy(
                    src_ref=x_hbm.at[pl.ds(p * m_per, m_per), :],
                    dst_ref=recv_buf.at[my],
                    send_sem=send_sems.at[d - 1],
                    recv_sem=recv_sems.at[my],
                    device_id=(p,),
                    device_id_type=pl.DeviceIdType.MESH,
                ).start()

        @pl.when(jnp.logical_and(j > 0, n == 0))
        def _():
            pltpu.make_async_remote_copy(
                src_ref=recv_buf.at[s],
                dst_ref=recv_buf.at[s],
                send_sem=send_sems.at[0],
                recv_sem=recv_sems.at[s],
                device_id=(my,),
                device_id_type=pl.DeviceIdType.MESH,
            ).wait_recv()

        nsl = pl.ds(n * N_TILE, N_TILE)

        @pl.when(j == 0)
        def _():
            out_ref[:, nsl] = jnp.dot(
                x_loc[...], w_blk[...], preferred_element_type=jnp.float32
            )

        @pl.when(jnp.logical_and(j > 0, j < N_DEV - 1))
        def _():
            out_ref[:, nsl] = out_ref[:, nsl] + jnp.dot(
                recv_buf[s], w_blk[...], preferred_element_type=jnp.float32
            )

        @pl.when(j == N_DEV - 1)
        def _():
            acc = out_ref[:, nsl] + jnp.dot(
                recv_buf[s], w_blk[...], preferred_element_type=jnp.float32
            )
            out_ref[:, nsl] = jnp.maximum(acc, 0.0)

        @pl.when(jnp.logical_and(j == N_DEV - 1, n == nt - 1))
        def _():
            for d in range(1, N_DEV):
                p = (my + d) % N_DEV
                pltpu.make_async_remote_copy(
                    src_ref=x_hbm.at[pl.ds(p * m_per, m_per), :],
                    dst_ref=recv_buf.at[my],
                    send_sem=send_sems.at[d - 1],
                    recv_sem=recv_sems.at[my],
                    device_id=(p,),
                    device_id_type=pl.DeviceIdType.MESH,
                ).wait_send()

    grid_spec = pltpu.PrefetchScalarGridSpec(
        num_scalar_prefetch=1,
        grid=(N_DEV, nt),
        in_specs=[
            pl.BlockSpec(memory_space=pltpu.ANY),
            pl.BlockSpec((m_per, k_shard), lambda j, n, pr: (pr[0], 0)),
            pl.BlockSpec((k_shard, N_TILE), lambda j, n, pr: (pr[j], n)),
        ],
        out_specs=pl.BlockSpec((m_per, n_total), lambda j, n, pr: (0, 0)),
        scratch_shapes=[
            pltpu.VMEM((N_DEV, m_per, k_shard), jnp.float32),
            pltpu.SemaphoreType.DMA((N_DEV - 1,)),
            pltpu.SemaphoreType.DMA((N_DEV,)),
        ],
    )

    return pl.pallas_call(
        body,
        grid_spec=grid_spec,
        out_shape=jax.ShapeDtypeStruct((m_per, n_total), jnp.float32),
        compiler_params=pltpu.CompilerParams(
            dimension_semantics=("arbitrary", "arbitrary"),
        ),
    )(order, x, x, w_mat)


# baseline (device time: 208698 ns/iter reference)
import jax
import jax.numpy as jnp
from jax import lax
from jax.experimental import pallas as pl
from jax.experimental.pallas import tpu as pltpu

N_DEV = 4
N_TILE = 512


def kernel(x, w_mat):
    m_total, k_shard = x.shape
    k_total, n_total = w_mat.shape
    m_per = m_total // N_DEV
    nt = n_total // N_TILE

    me = lax.axis_index("i")
    order = jnp.stack(
        [me, (me + 1) % N_DEV, (me + 3) % N_DEV, (me + 2) % N_DEV]
    ).astype(jnp.int32)

    def body(pr, x_hbm, x_loc, w_blk, out_ref, recv_buf, send_sems, recv_sems):
        j = pl.program_id(0)
        n = pl.program_id(1)
        my = pr[0]
        s = pr[j]

        @pl.when(jnp.logical_and(j == 0, n == 0))
        def _():
            for d in range(1, N_DEV):
                p = (my + d) % N_DEV
                pltpu.make_async_remote_copy(
                    src_ref=x_hbm.at[pl.ds(p * m_per, m_per), :],
                    dst_ref=recv_buf.at[my],
                    send_sem=send_sems.at[d - 1],
                    recv_sem=recv_sems.at[my],
                    device_id=p,
                    device_id_type=pl.DeviceIdType.LOGICAL,
                ).start()

        @pl.when(jnp.logical_and(j > 0, n == 0))
        def _():
            pltpu.make_async_remote_copy(
                src_ref=recv_buf.at[s],
                dst_ref=recv_buf.at[s],
                send_sem=send_sems.at[0],
                recv_sem=recv_sems.at[s],
                device_id=my,
                device_id_type=pl.DeviceIdType.LOGICAL,
            ).wait_recv()

        nsl = pl.ds(n * N_TILE, N_TILE)

        @pl.when(j == 0)
        def _():
            out_ref[:, nsl] = jnp.dot(
                x_loc[...], w_blk[...], preferred_element_type=jnp.float32
            )

        @pl.when(jnp.logical_and(j > 0, j < N_DEV - 1))
        def _():
            out_ref[:, nsl] = out_ref[:, nsl] + jnp.dot(
                recv_buf[s], w_blk[...], preferred_element_type=jnp.float32
            )

        @pl.when(j == N_DEV - 1)
        def _():
            acc = out_ref[:, nsl] + jnp.dot(
                recv_buf[s], w_blk[...], preferred_element_type=jnp.float32
            )
            out_ref[:, nsl] = jnp.maximum(acc, 0.0)

        @pl.when(jnp.logical_and(j == N_DEV - 1, n == nt - 1))
        def _():
            for d in range(1, N_DEV):
                p = (my + d) % N_DEV
                pltpu.make_async_remote_copy(
                    src_ref=x_hbm.at[pl.ds(p * m_per, m_per), :],
                    dst_ref=recv_buf.at[my],
                    send_sem=send_sems.at[d - 1],
                    recv_sem=recv_sems.at[my],
                    device_id=p,
                    device_id_type=pl.DeviceIdType.LOGICAL,
                ).wait_send()

    grid_spec = pltpu.PrefetchScalarGridSpec(
        num_scalar_prefetch=1,
        grid=(N_DEV, nt),
        in_specs=[
            pl.BlockSpec(memory_space=pl.ANY),
            pl.BlockSpec((m_per, k_shard), lambda j, n, pr: (pr[0], 0)),
            pl.BlockSpec((k_shard, N_TILE), lambda j, n, pr: (pr[j], n)),
        ],
        out_specs=pl.BlockSpec((m_per, n_total), lambda j, n, pr: (0, 0)),
        scratch_shapes=[
            pltpu.VMEM((N_DEV, m_per, k_shard), jnp.float32),
            pltpu.SemaphoreType.DMA((N_DEV - 1,)),
            pltpu.SemaphoreType.DMA((N_DEV,)),
        ],
    )

    return pl.pallas_call(
        body,
        grid_spec=grid_spec,
        out_shape=jax.ShapeDtypeStruct((m_per, n_total), jnp.float32),
        compiler_params=pltpu.CompilerParams(
            dimension_semantics=("arbitrary", "arbitrary"),
            vmem_limit_bytes=64 * 1024 * 1024,
        ),
    )(order, x, x, w_mat)
